# baseline (speedup 1.0000x reference)
import functools

import jax
import jax.numpy as jnp
from jax import lax
from jax.experimental import pallas as pl
from jax.experimental.pallas import tpu as pltpu
from jax.experimental.pallas import tpu_sc as plsc

TV = 2048


@functools.cache
def _sc_gather(num_rows: int, vocab: int, embed: int):
    info = plsc.get_sparse_core_info()
    nw = info.num_cores * info.num_subcores
    bpw = num_rows // nw
    chunk = 128
    nch = (bpw + chunk - 1) // chunk
    mesh = plsc.VectorSubcoreMesh(core_axis_name="c", subcore_axis_name="s")

    @functools.partial(
        pl.kernel,
        mesh=mesh,
        compiler_params=pltpu.CompilerParams(use_tc_tiling_on_sc=False),
        out_type=jax.ShapeDtypeStruct((num_rows, embed), jnp.float32),
        scratch_types=[
            pltpu.VMEM((bpw,), jnp.int32),
            pltpu.VMEM((bpw, embed), jnp.float32),
            pltpu.SemaphoreType.DMA,
        ],
    )
    def gather_kernel(idx_hbm, table_hbm, out_hbm, idx_v, rows_v, sem):
        wid = lax.axis_index("s") * info.num_cores + lax.axis_index("c")
        base = wid * bpw
        pltpu.sync_copy(idx_hbm.at[pl.ds(base, bpw)], idx_v)
        copies = []
        for c in range(nch):
            off = c * chunk
            sz = min(chunk, bpw - off)
            copies.append(
                pltpu.async_copy(
                    table_hbm.at[idx_v.at[pl.ds(off, sz)]],
                    rows_v.at[pl.ds(off, sz)],
                    sem,
                )
            )
        for cp in copies:
            cp.wait()
        pltpu.sync_copy(rows_v, out_hbm.at[pl.ds(base, bpw)])

    return gather_kernel


def _fc1(x, W1, b1):
    batch, feat = x.shape
    hidden = W1.shape[1]

    def body(x_ref, w_ref, b_ref, h_ref):
        acc = jnp.dot(
            x_ref[...].astype(jnp.bfloat16),
            w_ref[...].astype(jnp.bfloat16),
            preferred_element_type=jnp.float32,
        )
        h_ref[...] = jnp.maximum(acc + b_ref[...], 0.0).astype(jnp.bfloat16)

    return pl.pallas_call(
        body,
        out_shape=jax.ShapeDtypeStruct((batch, hidden), jnp.bfloat16),
    )(x, W1, b1.reshape(1, hidden))


def _lse_pass(h_bf, W2, b2):
    batch, hidden = h_bf.shape
    vocab = W2.shape[1]
    nj = (vocab + TV - 1) // TV

    def body(h_ref, w_ref, b_ref, lse_ref, m_ref, s_ref):
        j = pl.program_id(0)

        @pl.when(j == 0)
        def _():
            m_ref[...] = jnp.full((batch, 1), -jnp.inf, jnp.float32)
            s_ref[...] = jnp.zeros((batch, 1), jnp.float32)

        logits = (
            jnp.dot(
                h_ref[...],
                w_ref[...].astype(jnp.bfloat16),
                preferred_element_type=jnp.float32,
            )
            + b_ref[...]
        )
        col = j * TV + lax.broadcasted_iota(jnp.int32, (1, TV), 1)
        logits = jnp.where(col < vocab, logits, -jnp.inf)
        m_old = m_ref[...]
        m_new = jnp.maximum(m_old, jnp.max(logits, axis=1, keepdims=True))
        s_new = s_ref[...] * jnp.exp(m_old - m_new) + jnp.sum(
            jnp.exp(logits - m_new), axis=1, keepdims=True
        )
        m_ref[...] = m_new
        s_ref[...] = s_new
        lse_ref[...] = m_new + jnp.log(s_new)

    return pl.pallas_call(
        body,
        grid=(nj,),
        in_specs=[
            pl.BlockSpec((batch, hidden), lambda j: (0, 0)),
            pl.BlockSpec((hidden, TV), lambda j: (0, j)),
            pl.BlockSpec((1, TV), lambda j: (0, j)),
        ],
        out_specs=pl.BlockSpec((batch, 1), lambda j: (0, 0)),
        out_shape=jax.ShapeDtypeStruct((batch, 1), jnp.float32),
        scratch_shapes=[
            pltpu.VMEM((batch, 1), jnp.float32),
            pltpu.VMEM((batch, 1), jnp.float32),
        ],
    )(h_bf, W2, b2.reshape(1, vocab))


def _out_pass(h_bf, W2, b2, lse):
    batch, hidden = h_bf.shape
    vocab = W2.shape[1]
    nj = (vocab + TV - 1) // TV

    def body(h_ref, w_ref, b_ref, lse_ref, o_ref):
        logits = (
            jnp.dot(
                h_ref[...],
                w_ref[...].astype(jnp.bfloat16),
                preferred_element_type=jnp.float32,
            )
            + b_ref[...]
        )
        o_ref[...] = logits - lse_ref[...]

    return pl.pallas_call(
        body,
        grid=(nj,),
        in_specs=[
            pl.BlockSpec((batch, hidden), lambda j: (0, 0)),
            pl.BlockSpec((hidden, TV), lambda j: (0, j)),
            pl.BlockSpec((1, TV), lambda j: (0, j)),
            pl.BlockSpec((batch, 1), lambda j: (0, 0)),
        ],
        out_specs=pl.BlockSpec((batch, TV), lambda j: (0, j)),
        out_shape=jax.ShapeDtypeStruct((batch, vocab), jnp.float32),
    )(h_bf, W2, b2.reshape(1, vocab), lse)


def kernel(inputs, emb, W1, b1, W2, b2):
    batch, ctx2 = inputs.shape
    vocab, embed = emb.shape
    idx = inputs.astype(jnp.int32).reshape(-1)
    gathered = _sc_gather(batch * ctx2, vocab, embed)(idx, emb)
    x = gathered.reshape(batch, ctx2 * embed)
    h = _fc1(x, W1, b1)
    lse = _lse_pass(h, W2, b2)
    return _out_pass(h, W2, b2, lse)
